# baseline (speedup 1.0000x reference)
import functools

import jax
import jax.numpy as jnp
from jax import lax
from jax.experimental import pallas as pl
from jax.experimental.pallas import tpu as pltpu
from jax.experimental.pallas import tpu_sc as plsc

LANE = 128
SUB = 8
NBUF = 2


@functools.partial(jax.jit, static_argnums=(2, 3, 4, 5, 6))
def _sc_gather(idx2d, table, f, b, d, nc, ns):
    nw = nc * ns
    n_units = idx2d.shape[0]
    units_w = n_units // nw
    nb = b // LANE
    cbs = d // SUB
    mesh = plsc.VectorSubcoreMesh(
        core_axis_name="c", subcore_axis_name="s",
        num_cores=nc, num_subcores=ns)

    @functools.partial(
        pl.kernel,
        out_type=jax.ShapeDtypeStruct((f, cbs, nb, SUB, LANE), jnp.float32),
        mesh=mesh,
        scratch_types=(
            [pltpu.VMEM((units_w, LANE), jnp.int32)]
            + [pltpu.VMEM((LANE, d), jnp.float32)] * NBUF
            + [pltpu.VMEM((cbs, SUB, LANE), jnp.float32)] * NBUF
            + [pltpu.SemaphoreType.DMA] * (2 * NBUF)
        ),
        compiler_params=pltpu.CompilerParams(
            use_tc_tiling_on_sc=False, needs_layout_passes=False),
    )
    def k(table_hbm, idx_hbm, r_hbm, idx_v, chunk0, chunk1, tb0, tb1,
          sg0, sg1, so0, so1):
        wid = lax.axis_index("s") * nc + lax.axis_index("c")
        u0 = wid * units_w
        chunk = (chunk0, chunk1)
        tbuf = (tb0, tb1)
        sg = (sg0, sg1)
        so = (so0, so1)

        pltpu.sync_copy(idx_hbm.at[pl.ds(pl.multiple_of(u0, 8), units_w)],
                        idx_v)

        iota = lax.iota(jnp.int32, 16)

        def issue(u, s):
            pltpu.async_copy(table_hbm.at[idx_v.at[u]], chunk[s], sg[s])

        def consume(u, s):
            pltpu.make_async_copy(
                table_hbm.at[idx_v.at[0]], chunk[s], sg[s]).wait()
            for cb in range(cbs):
                for cl in range(SUB):
                    col = jnp.full((16,), cb * SUB + cl, jnp.int32)
                    for l0 in range(0, LANE, 16):
                        vals = plsc.load_gather(chunk[s], [l0 + iota, col])
                        tbuf[s][cb, cl, pl.ds(l0, 16)] = vals
            fu = (u0 + u) // nb
            ib = (u0 + u) % nb
            for cb in range(cbs):
                pltpu.async_copy(tbuf[s].at[cb], r_hbm.at[fu, cb, ib], so[s])

        def wait_out(s):
            for cb in range(cbs):
                pltpu.make_async_copy(
                    tbuf[s].at[cb], r_hbm.at[0, cb, 0], so[s]).wait()

        issue(0, 0)
        issue(1, 1)

        def body(i, _):
            c0 = 2 * i
            consume(c0, 0)

            @pl.when(c0 + 2 < units_w)
            def _():
                wait_out(0)
                issue(c0 + 2, 0)

            consume(c0 + 1, 1)

            @pl.when(c0 + 3 < units_w)
            def _():
                wait_out(1)
                issue(c0 + 3, 1)

            return 0

        if units_w % 2:
            lax.fori_loop(0, (units_w - 1) // 2, body, 0)
            consume(units_w - 1, 0)
        else:
            lax.fori_loop(0, units_w // 2 - 1, body, 0)
            consume(units_w - 2, 0)
            consume(units_w - 1, 1)
        wait_out(0)
        wait_out(1)

    return k(table, idx2d)


def kernel(input_indices, table):
    b, f = input_indices.shape
    v, d = table.shape
    idx2d = input_indices.T.reshape(f * b // LANE, LANE).astype(jnp.int32)
    info = plsc.get_sparse_core_info()
    r = _sc_gather(idx2d, table, f, b, d, info.num_cores, info.num_subcores)
    return r.transpose(2, 4, 0, 1, 3).reshape(b, f, d)


# speedup vs baseline: 1.3954x; 1.3954x over previous
import functools

import jax
import jax.numpy as jnp
from jax import lax
from jax.experimental import pallas as pl
from jax.experimental.pallas import tpu as pltpu
from jax.experimental.pallas import tpu_sc as plsc

LANE = 128
SUB = 8
NBUF = 2


@functools.partial(jax.jit, static_argnums=(2, 3, 4, 5, 6))
def _sc_gather(idx2d, table, f, b, d, nc, ns):
    nw = nc * ns
    n_units = idx2d.shape[0]
    units_w = n_units // nw
    nb = b // LANE
    cbs = d // SUB
    mesh = plsc.VectorSubcoreMesh(
        core_axis_name="c", subcore_axis_name="s",
        num_cores=nc, num_subcores=ns)

    @functools.partial(
        pl.kernel,
        out_type=jax.ShapeDtypeStruct((f, cbs, nb, SUB, LANE), jnp.float32),
        mesh=mesh,
        scratch_types=(
            [pltpu.VMEM((units_w, LANE), jnp.int32)]
            + [pltpu.VMEM((LANE, d), jnp.float32)] * NBUF
            + [pltpu.VMEM((d, LANE + 1), jnp.float32)] * NBUF
            + [pltpu.SemaphoreType.DMA] * (2 * NBUF)
        ),
        compiler_params=pltpu.CompilerParams(
            use_tc_tiling_on_sc=False, needs_layout_passes=False),
    )
    def k(table_hbm, idx_hbm, r_hbm, idx_v, chunk0, chunk1, tb0, tb1,
          sg0, sg1, so0, so1):
        wid = lax.axis_index("s") * nc + lax.axis_index("c")
        u0 = wid * units_w
        chunk = (chunk0, chunk1)
        tbuf = (tb0, tb1)
        sg = (sg0, sg1)
        so = (so0, so1)

        pltpu.sync_copy(idx_hbm.at[pl.ds(pl.multiple_of(u0, 8), units_w)],
                        idx_v)

        iota = lax.iota(jnp.int32, 16)

        def issue(u, s):
            pltpu.async_copy(table_hbm.at[idx_v.at[u]], chunk[s], sg[s])

        def consume(u, s):
            pltpu.make_async_copy(
                table_hbm.at[idx_v.at[0]], chunk[s], sg[s]).wait()
            for j in range(LANE):
                colj = jnp.full((16,), j, jnp.int32)
                for c0 in range(0, d, 16):
                    vals = chunk[s][j, pl.ds(c0, 16)]
                    plsc.store_scatter(tbuf[s], [c0 + iota, colj], vals)
            fu = (u0 + u) // nb
            ib = (u0 + u) % nb
            for cb in range(cbs):
                pltpu.async_copy(
                    tbuf[s].at[pl.ds(cb * SUB, SUB), pl.ds(0, LANE)],
                    r_hbm.at[fu, cb, ib], so[s])

        def wait_out(s):
            for cb in range(cbs):
                pltpu.make_async_copy(
                    tbuf[s].at[pl.ds(cb * SUB, SUB), pl.ds(0, LANE)],
                    r_hbm.at[0, cb, 0], so[s]).wait()

        issue(0, 0)
        issue(1, 1)

        def body(i, _):
            c0 = 2 * i
            consume(c0, 0)

            @pl.when(c0 + 2 < units_w)
            def _():
                wait_out(0)
                issue(c0 + 2, 0)

            consume(c0 + 1, 1)

            @pl.when(c0 + 3 < units_w)
            def _():
                wait_out(1)
                issue(c0 + 3, 1)

            return 0

        if units_w % 2:
            lax.fori_loop(0, (units_w - 1) // 2, body, 0)
            consume(units_w - 1, 0)
        else:
            lax.fori_loop(0, units_w // 2 - 1, body, 0)
            consume(units_w - 2, 0)
            consume(units_w - 1, 1)
        wait_out(0)
        wait_out(1)

    return k(table, idx2d)


def kernel(input_indices, table):
    b, f = input_indices.shape
    v, d = table.shape
    idx2d = input_indices.T.reshape(f * b // LANE, LANE).astype(jnp.int32)
    info = plsc.get_sparse_core_info()
    r = _sc_gather(idx2d, table, f, b, d, info.num_cores, info.num_subcores)
    return r.transpose(2, 4, 0, 1, 3).reshape(b, f, d)
